# Initial kernel scaffold; baseline (speedup 1.0000x reference)
import jax
import jax.numpy as jnp
from jax.experimental import pallas as pl


def kernel(x_user, x_item, ei_u2i, ei_i2u, params):
    raise NotImplementedError("write your pallas kernel here")


# speedup vs baseline: 20.9960x; 20.9960x over previous
import functools

import jax
import jax.numpy as jnp
import numpy as np
from jax import lax
from jax.experimental import pallas as pl
from jax.experimental.pallas import tpu as pltpu
from jax.experimental.pallas import tpu_sc as plsc

H = 2
DH = 64
HID = H * DH
KV = 2 * HID

_NC = 2
_NS = 16
_NW = _NC * _NS
_CH = 64


_BR = 512


def _grid(n):
    return (n + _BR - 1) // _BR


def _tc_proj1_body(x_ref, w_ref, b_ref, wkv_ref, wq_ref, h_ref, kv_ref, q_ref):
    h = jnp.maximum(
        jnp.dot(x_ref[...], w_ref[...], preferred_element_type=jnp.float32)
        + b_ref[...],
        0.0,
    )
    h_ref[...] = h
    kv_ref[...] = jnp.dot(h, wkv_ref[...], preferred_element_type=jnp.float32)
    q_ref[...] = jnp.dot(h, wq_ref[...], preferred_element_type=jnp.float32)


def _tc_proj1(x, w, b, wkv, wq):
    n, d = x.shape
    return pl.pallas_call(
        _tc_proj1_body,
        grid=(_grid(n),),
        in_specs=[
            pl.BlockSpec((_BR, d), lambda i: (i, 0)),
            pl.BlockSpec((d, HID), lambda i: (0, 0)),
            pl.BlockSpec((1, HID), lambda i: (0, 0)),
            pl.BlockSpec((HID, KV), lambda i: (0, 0)),
            pl.BlockSpec((HID, HID), lambda i: (0, 0)),
        ],
        out_specs=[
            pl.BlockSpec((_BR, HID), lambda i: (i, 0)),
            pl.BlockSpec((_BR, KV), lambda i: (i, 0)),
            pl.BlockSpec((_BR, HID), lambda i: (i, 0)),
        ],
        out_shape=[
            jax.ShapeDtypeStruct((n, HID), jnp.float32),
            jax.ShapeDtypeStruct((n, KV), jnp.float32),
            jax.ShapeDtypeStruct((n, HID), jnp.float32),
        ],
    )(x, w, b, wkv, wq)


def _tc_proj_body(x_ref, wkv_ref, wq_ref, kv_ref, q_ref):
    x = x_ref[...]
    kv_ref[...] = jnp.dot(x, wkv_ref[...], preferred_element_type=jnp.float32)
    q_ref[...] = jnp.dot(x, wq_ref[...], preferred_element_type=jnp.float32)


def _tc_proj(x, wkv, wq):
    n, _ = x.shape
    return pl.pallas_call(
        _tc_proj_body,
        grid=(_grid(n),),
        in_specs=[
            pl.BlockSpec((_BR, HID), lambda i: (i, 0)),
            pl.BlockSpec((HID, KV), lambda i: (0, 0)),
            pl.BlockSpec((HID, HID), lambda i: (0, 0)),
        ],
        out_specs=[
            pl.BlockSpec((_BR, KV), lambda i: (i, 0)),
            pl.BlockSpec((_BR, HID), lambda i: (i, 0)),
        ],
        out_shape=[
            jax.ShapeDtypeStruct((n, KV), jnp.float32),
            jax.ShapeDtypeStruct((n, HID), jnp.float32),
        ],
    )(x, wkv, wq)


def _tc_post_body(outp_ref, den_ref, x_ref, wa_ref, beta_ref, o_ref):
    s = outp_ref[0] + outp_ref[1]
    den = den_ref[...]
    d0 = den[:, 0:1] + 1e-16
    d1 = den[:, 1:2] + 1e-16
    br = s.shape[0]
    div = jnp.concatenate(
        [jnp.broadcast_to(d0, (br, DH)), jnp.broadcast_to(d1, (br, DH))], axis=1
    )
    agg = s / div
    o = jnp.dot(jax.nn.gelu(agg), wa_ref[...], preferred_element_type=jnp.float32)
    beta = beta_ref[0]
    o_ref[...] = beta * o + (1.0 - beta) * x_ref[...]


def _tc_post(outp, den, x, wa, beta, out_dim):
    n, _ = x.shape
    return pl.pallas_call(
        _tc_post_body,
        grid=(_grid(n),),
        in_specs=[
            pl.BlockSpec((2, _BR, HID), lambda i: (0, i, 0)),
            pl.BlockSpec((_BR, 16), lambda i: (i, 0)),
            pl.BlockSpec((_BR, HID), lambda i: (i, 0)),
            pl.BlockSpec((HID, out_dim), lambda i: (0, 0)),
            pl.BlockSpec(memory_space=pltpu.SMEM),
        ],
        out_specs=pl.BlockSpec((_BR, out_dim), lambda i: (i, 0)),
        out_shape=jax.ShapeDtypeStruct((n, out_dim), jnp.float32),
    )(outp, den, x, wa, beta)


def _rows_per_subcore(nd):
    return 8 * ((nd + _NS * 8 - 1) // (_NS * 8))


def _den_rows(nd):
    return 8 * _NS * ((_rows_per_subcore(nd) * 2 + 8 * _NS - 1) // (8 * _NS))


def _sc_edge_body(nd, ne,
                  kv_hbm, q_hbm, si_hbm, di_hbm, ps_hbm, z128_hbm,
                  outp_hbm, denp_hbm,
                  si_v, di_v, rdi_v, kv_v, q_v, exd_v, ps_v,
                  out_sh, den_sh, sem_a, sem_b):
    c = lax.axis_index("c")
    s = lax.axis_index("s")
    wid = s * _NC + c

    rows = _rows_per_subcore(nd)
    r0 = s * rows
    drows = _den_rows(nd) // _NS
    pltpu.sync_copy(z128_hbm.at[pl.ds(r0, rows)], out_sh.at[pl.ds(r0, rows)])
    pltpu.sync_copy(z128_hbm.at[pl.ds(s * drows, drows)],
                    den_sh.at[pl.ds(s * drows, drows)])
    pltpu.sync_copy(ps_hbm, ps_v)
    plsc.subcore_barrier()

    nchunks = ne // _CH
    iters = (nchunks + _NW - 1) // _NW
    lane = lax.broadcasted_iota(jnp.int32, (16,), 0)
    ps0 = ps_v[0, :]
    ps1 = ps_v[1, :]

    gdn = lax.GatherDimensionNumbers(
        offset_dims=(), collapsed_slice_dims=(0,), start_index_map=(0,))

    def allsum(v):
        for sh in (8, 4, 2, 1):
            perm = lax.gather(
                v, (lane ^ sh)[:, None], gdn, (1,),
                mode=lax.GatherScatterMode.PROMISE_IN_BOUNDS)
            v = v + perm
        return v

    def chunk(i, _):
        cid = wid + i * _NW

        @pl.when(cid < nchunks)
        def _():
            off = cid * _CH
            pltpu.sync_copy(si_hbm.at[pl.ds(off, _CH)], si_v)
            pltpu.sync_copy(di_hbm.at[pl.ds(off, _CH)], di_v)
            cp_kv = pltpu.async_copy(kv_hbm.at[si_v], kv_v, sem_a)
            cp_q = pltpu.async_copy(q_hbm.at[di_v], q_v, sem_b)
            for g in range(_CH // 16):
                rdi_v[pl.ds(g * 16, 16)] = lax.shift_right_logical(
                    di_v[pl.ds(g * 16, 16)], 3)
            cp_kv.wait()
            cp_q.wait()

            def edge(e, _):
                acc0 = kv_v[e, pl.ds(0, 16)] * q_v[e, pl.ds(0, 16)]
                acc1 = kv_v[e, pl.ds(DH, 16)] * q_v[e, pl.ds(DH, 16)]
                for j in range(1, 4):
                    acc0 += kv_v[e, pl.ds(j * 16, 16)] * q_v[e, pl.ds(j * 16, 16)]
                    acc1 += (kv_v[e, pl.ds(DH + j * 16, 16)]
                             * q_v[e, pl.ds(DH + j * 16, 16)])
                ex0 = jnp.exp(allsum(acc0) * ps0)
                ex1 = jnp.exp(allsum(acc1) * ps1)
                ex = jnp.where(lane == 0, ex0, jnp.where(lane == 1, ex1, 0.0))
                zero16 = jnp.zeros((16,), jnp.float32)
                g = e // 16
                dv = di_v[pl.ds(g * 16, 16)]
                lsel = lane * 0 + (e - g * 16)
                dsplat = lax.gather(
                    dv, lsel[:, None], gdn, (1,),
                    mode=lax.GatherScatterMode.PROMISE_IN_BOUNDS)
                bf = (dsplat & 7).astype(jnp.float32)
                for j in range(8):
                    m = 1.0 - jnp.minimum(jnp.abs(bf - float(j)), 1.0)
                    exd_v[e, pl.ds(j * 16, 16)] = ex * m
                for j in range(8):
                    w = ex0 if j < 4 else ex1
                    q_v[e, pl.ds(j * 16, 16)] = (
                        kv_v[e, pl.ds(HID + j * 16, 16)] * w)
                return 0

            lax.fori_loop(0, _CH, edge, 0)
            pltpu.sync_copy(q_v, out_sh.at[di_v], add=True)
            pltpu.sync_copy(exd_v, den_sh.at[rdi_v], add=True)

        return 0

    lax.fori_loop(0, iters, chunk, 0)
    plsc.subcore_barrier()
    pltpu.sync_copy(out_sh.at[pl.ds(r0, rows)], outp_hbm.at[c, pl.ds(r0, rows)])
    pltpu.sync_copy(den_sh.at[pl.ds(s * drows, drows)],
                    denp_hbm.at[c, pl.ds(s * drows, drows)])


def _sc_edge(kv, q, si, di, ps, z128):
    nd = q.shape[0]
    ne = si.shape[0]
    np_pad = _rows_per_subcore(nd) * _NS
    dr = _den_rows(nd)
    mesh = plsc.VectorSubcoreMesh(core_axis_name="c", subcore_axis_name="s")
    f = pl.kernel(
        functools.partial(_sc_edge_body, nd, ne),
        out_type=[
            jax.ShapeDtypeStruct((_NC, np_pad, HID), jnp.float32),
            jax.ShapeDtypeStruct((_NC, dr, HID), jnp.float32),
        ],
        mesh=mesh,
        scratch_types=[
            pltpu.VMEM((_CH,), jnp.int32),
            pltpu.VMEM((_CH,), jnp.int32),
            pltpu.VMEM((_CH,), jnp.int32),
            pltpu.VMEM((_CH, KV), jnp.float32),
            pltpu.VMEM((_CH, HID), jnp.float32),
            pltpu.VMEM((_CH, HID), jnp.float32),
            pltpu.VMEM((2, 16), jnp.float32),
            pltpu.VMEM_SHARED((np_pad, HID), jnp.float32),
            pltpu.VMEM_SHARED((dr, HID), jnp.float32),
            pltpu.SemaphoreType.DMA,
            pltpu.SemaphoreType.DMA,
        ],
    )
    outp, denp = f(kv, q, si, di, ps, z128)
    den = denp.sum(axis=0).reshape(dr * 8, 16)[:np_pad]
    return outp, den


def _fused_kv_weights(node_p, rel_p):
    wk = node_p["Wk"]
    wv = node_p["Wv"]
    k_cols = [wk[:, h * DH:(h + 1) * DH] @ rel_p["a"][h] for h in range(H)]
    v_cols = [wv[:, h * DH:(h + 1) * DH] @ rel_p["m"][h] for h in range(H)]
    return jnp.concatenate(k_cols + v_cols, axis=1)


def _pscale(rel_p):
    return jnp.broadcast_to(
        (rel_p["p"] / np.sqrt(DH))[:, None], (H, 16)).astype(jnp.float32)


def kernel(x_user, x_item, ei_u2i, ei_i2u, params):
    n_user = x_user.shape[0]
    n_item = x_item.shape[0]
    np_pad = _rows_per_subcore(max(n_user, n_item)) * _NS
    z128 = jnp.zeros((np_pad, HID), jnp.float32)

    si_u2i = ei_u2i[0]
    di_u2i = ei_u2i[1]
    si_i2u = ei_i2u[0]
    di_i2u = ei_i2u[1]

    lp0, lp1 = params["layers"]

    lin_u = params["lin"]["user"]
    lin_i = params["lin"]["item"]
    h_u, kv_u, q_u = _tc_proj1(
        x_user, lin_u["W"], lin_u["b"].reshape(1, HID),
        _fused_kv_weights(lp0["node"]["user"], lp0["rel"]["u2i"]),
        lp0["node"]["user"]["Wq"])
    h_i, kv_i, q_i = _tc_proj1(
        x_item, lin_i["W"], lin_i["b"].reshape(1, HID),
        _fused_kv_weights(lp0["node"]["item"], lp0["rel"]["i2u"]),
        lp0["node"]["item"]["Wq"])

    outp_i, denp_i = _sc_edge(kv_u, q_i, si_u2i, di_u2i,
                              _pscale(lp0["rel"]["u2i"]), z128)
    outp_u, denp_u = _sc_edge(kv_i, q_u, si_i2u, di_i2u,
                              _pscale(lp0["rel"]["i2u"]), z128)

    beta_u0 = jax.nn.sigmoid(lp0["node"]["user"]["skip"]).reshape(1)
    beta_i0 = jax.nn.sigmoid(lp0["node"]["item"]["skip"]).reshape(1)
    x_u1 = _tc_post(outp_u, denp_u, h_u, lp0["node"]["user"]["Wa"], beta_u0, HID)
    x_i1 = _tc_post(outp_i, denp_i, h_i, lp0["node"]["item"]["Wa"], beta_i0, HID)

    kv_u2, q_u2 = _tc_proj(
        x_u1, _fused_kv_weights(lp1["node"]["user"], lp1["rel"]["u2i"]),
        lp1["node"]["user"]["Wq"])
    kv_i2, q_i2 = _tc_proj(
        x_i1, _fused_kv_weights(lp1["node"]["item"], lp1["rel"]["i2u"]),
        lp1["node"]["item"]["Wq"])

    outp_i2, denp_i2 = _sc_edge(kv_u2, q_i2, si_u2i, di_u2i,
                                _pscale(lp1["rel"]["u2i"]), z128)
    outp_u2, denp_u2 = _sc_edge(kv_i2, q_u2, si_i2u, di_i2u,
                                _pscale(lp1["rel"]["i2u"]), z128)

    beta_u1 = jax.nn.sigmoid(lp1["node"]["user"]["skip"]).reshape(1)
    beta_i1 = jax.nn.sigmoid(lp1["node"]["item"]["skip"]).reshape(1)
    out_u = _tc_post(outp_u2, denp_u2, x_u1, lp1["node"]["user"]["Wa"], beta_u1, HID)
    out_i = _tc_post(outp_i2, denp_i2, x_i1, lp1["node"]["item"]["Wa"], beta_i1, HID)
    return (out_u, out_i)
